# baseline (speedup 1.0000x reference)
import jax
import jax.numpy as jnp
from jax.experimental import pallas as pl
from jax.experimental.pallas import tpu as pltpu

_BF = jnp.bfloat16
_F32 = jnp.float32


def _half(xt, w1s, w2s, whs, bht):
    tbl = xt.shape[1]
    x_hi = xt.astype(_BF)
    x_lo = (xt - x_hi.astype(_F32)).astype(_BF)
    ones2 = jnp.ones((2, tbl), _BF)

    xs = jnp.concatenate([x_hi, x_hi, x_lo, ones2], axis=0)
    h1 = jnp.dot(w1s, xs, preferred_element_type=_F32)
    h1 = jnp.maximum(h1.astype(_BF), _BF(0))

    hs = jnp.concatenate([h1, ones2], axis=0)
    h2 = jnp.dot(w2s, hs, preferred_element_type=_F32)
    h2 = jnp.maximum(h2.astype(_BF), _BF(0))

    y = jnp.dot(whs, h2, preferred_element_type=_F32) + bht

    mu = 2.0 * jnp.tanh(y[0:1, :])
    sd = jnp.logaddexp(y[1:2, :], 0.0) + 0.001
    return mu, sd


def _policy_kernel(xt_ref, w1s_ref, w2s_ref, whs_ref, bht_ref, mu_ref, sd_ref):
    xt = xt_ref[...]
    h = xt.shape[1] // 2
    w1s, w2s = w1s_ref[...], w2s_ref[...]
    whs, bht = whs_ref[...], bht_ref[...]
    mu_a, sd_a = _half(xt[:, :h], w1s, w2s, whs, bht)
    mu_b, sd_b = _half(xt[:, h:], w1s, w2s, whs, bht)
    mu_ref[:, :h] = mu_a
    mu_ref[:, h:] = mu_b
    sd_ref[:, :h] = sd_a
    sd_ref[:, h:] = sd_b


def _hi_lo(a):
    hi = a.astype(_BF)
    lo = (a - hi.astype(_F32)).astype(_BF)
    return hi, lo


def _round_up(n, m):
    return ((n + m - 1) // m) * m


def kernel(x, w1, b1, w2, b2, wmu, bmu, wsd, bsd, *, tile_b=16384):
    B, F = x.shape
    assert F == 3

    tbl = max(128, min(_round_up(tile_b, 128), _round_up(B, 128)))
    Bp = pl.cdiv(B, tbl) * tbl
    xp = jnp.pad(x, ((0, Bp - B), (0, 0))) if Bp != B else x
    xt = xp.T

    w1_hi, w1_lo = _hi_lo(w1.T.astype(_F32))
    b1_hi, b1_lo = _hi_lo(b1.reshape(-1, 1).astype(_F32))
    w1s = jnp.concatenate([w1_hi, w1_lo, w1_hi, b1_hi, b1_lo], axis=1)

    b2_hi, b2_lo = _hi_lo(b2.reshape(-1, 1).astype(_F32))
    w2s = jnp.concatenate(
        [w2.T.astype(_BF), b2_hi, b2_lo], axis=1)

    whs = jnp.concatenate([wmu, wsd], axis=1).T.astype(_BF)
    bht = jnp.concatenate([bmu, bsd], axis=1).T.astype(_F32)

    batch_map = lambda i: (0, i)
    const_map = lambda i: (0, 0)

    weight_bytes = 2 * (64 * 11 + 256 * 66 + 2 * 256) + 4 * 2
    cost = pl.CostEstimate(
        flops=2 * Bp * (3 * 64 + 64 * 256 + 256 * 2),
        transcendentals=2 * Bp,
        bytes_accessed=20 * Bp + weight_bytes,
    )

    mu, sd = pl.pallas_call(
        _policy_kernel,
        out_shape=(
            jax.ShapeDtypeStruct((1, Bp), _F32),
            jax.ShapeDtypeStruct((1, Bp), _F32),
        ),
        grid=(Bp // tbl,),
        in_specs=[
            pl.BlockSpec((3, tbl), batch_map),
            pl.BlockSpec((64, 11), const_map),
            pl.BlockSpec((256, 66), const_map),
            pl.BlockSpec((2, 256), const_map),
            pl.BlockSpec((2, 1), const_map),
        ],
        out_specs=(
            pl.BlockSpec((1, tbl), batch_map),
            pl.BlockSpec((1, tbl), batch_map),
        ),
        compiler_params=pltpu.CompilerParams(
            dimension_semantics=("parallel",),
        ),
        cost_estimate=cost,
    )(xt, w1s, w2s, whs, bht)

    return mu.reshape(Bp, 1)[:B], sd.reshape(Bp, 1)[:B]


# speedup vs baseline: 1.0145x; 1.0145x over previous
import jax
import jax.numpy as jnp
from jax.experimental import pallas as pl
from jax.experimental.pallas import tpu as pltpu

_BF = jnp.bfloat16
_F32 = jnp.float32


def _policy_kernel(xt_ref, w1s_ref, w2s_ref, wht_ref, bht_ref, mu_ref, sd_ref):
    xt = xt_ref[...]
    tbl = xt.shape[1]
    x_hi = xt.astype(_BF)
    x_lo = (xt - x_hi.astype(_F32)).astype(_BF)
    ones2 = jnp.ones((2, tbl), _BF)

    xs = jnp.concatenate([x_hi, x_hi, x_lo, ones2], axis=0)
    h1 = jnp.dot(w1s_ref[...], xs, preferred_element_type=_F32)
    h1 = jnp.maximum(h1, 0.0)

    h_hi = h1.astype(_BF)
    hs = jnp.concatenate([h_hi, h_hi, ones2], axis=0)
    h2 = jnp.dot(w2s_ref[...], hs, preferred_element_type=_F32)
    h2 = jnp.maximum(h2, 0.0)

    y = jnp.dot(wht_ref[...], h2, preferred_element_type=_F32) + bht_ref[...]

    mu_ref[...] = 2.0 * jnp.tanh(y[0:1, :])
    sd_ref[...] = jnp.logaddexp(y[1:2, :], 0.0) + 0.001


def _hi_lo(a):
    hi = a.astype(_BF)
    lo = (a - hi.astype(_F32)).astype(_BF)
    return hi, lo


def _round_up(n, m):
    return ((n + m - 1) // m) * m


def kernel(x, w1, b1, w2, b2, wmu, bmu, wsd, bsd, *, tile_b=16384):
    B, F = x.shape
    assert F == 3

    tbl = max(128, min(_round_up(tile_b, 128), _round_up(B, 128)))
    Bp = pl.cdiv(B, tbl) * tbl
    xp = jnp.pad(x, ((0, Bp - B), (0, 0))) if Bp != B else x
    xt = xp.T

    w1_hi, w1_lo = _hi_lo(w1.T.astype(_F32))
    b1_hi, b1_lo = _hi_lo(b1.reshape(-1, 1).astype(_F32))
    w1s = jnp.concatenate([w1_hi, w1_lo, w1_hi, b1_hi, b1_lo], axis=1)

    w2_hi, w2_lo = _hi_lo(w2.T.astype(_F32))
    b2_hi, b2_lo = _hi_lo(b2.reshape(-1, 1).astype(_F32))
    w2s = jnp.concatenate([w2_hi, w2_lo, b2_hi, b2_lo], axis=1)

    wht = jnp.concatenate([wmu, wsd], axis=1).T.astype(_F32)
    bht = jnp.concatenate([bmu, bsd], axis=1).T.astype(_F32)

    batch_map = lambda i: (0, i)
    const_map = lambda i: (0, 0)

    weight_bytes = 2 * (64 * 11 + 256 * 130) + 4 * (2 * 256 + 2)
    cost = pl.CostEstimate(
        flops=2 * Bp * (3 * 64 + 64 * 256 + 256 * 2),
        transcendentals=2 * Bp,
        bytes_accessed=20 * Bp + weight_bytes,
    )

    mu, sd = pl.pallas_call(
        _policy_kernel,
        out_shape=(
            jax.ShapeDtypeStruct((1, Bp), _F32),
            jax.ShapeDtypeStruct((1, Bp), _F32),
        ),
        grid=(Bp // tbl,),
        in_specs=[
            pl.BlockSpec((3, tbl), batch_map),
            pl.BlockSpec((64, 11), const_map),
            pl.BlockSpec((256, 130), const_map),
            pl.BlockSpec((2, 256), const_map),
            pl.BlockSpec((2, 1), const_map),
        ],
        out_specs=(
            pl.BlockSpec((1, tbl), batch_map),
            pl.BlockSpec((1, tbl), batch_map),
        ),
        compiler_params=pltpu.CompilerParams(
            dimension_semantics=("parallel",),
        ),
        cost_estimate=cost,
    )(xt, w1s, w2s, wht, bht)

    return mu.reshape(Bp, 1)[:B], sd.reshape(Bp, 1)[:B]
